# baseline (speedup 1.0000x reference)
import functools

import jax
import jax.numpy as jnp
from jax import lax
from jax.experimental import pallas as pl
from jax.experimental.pallas import tpu as pltpu
from jax.experimental.pallas import tpu_sc as plsc

N = 50000
D = 512
S = 512

_FMIN = jnp.finfo(jnp.float32).min

_RA = 2000
_NBA = N // _RA

_NC = 2
_NS = 16
_NPAD = 51200
_STAT_PER_T = _NPAD // _NS
_ALPHA_PER_T = _NPAD // (_NC * _NS)

_RSH = 8
_CG = 4
_CGW = D // _CG
_CH = 128
_PER_SH = _NPAD // _RSH
_NCH = _PER_SH // _CH


def _gate_body(nodes_ref, wg_ref, bg_ref, x_ref):
    x_ref[...] = jnp.dot(nodes_ref[...], wg_ref[...],
                         preferred_element_type=jnp.float32) + bg_ref[0, 0]


def _gate_call(nodes, w_gate, b_gate):
    return pl.pallas_call(
        _gate_body,
        grid=(_NBA,),
        in_specs=[
            pl.BlockSpec((_RA, D), lambda i: (i, 0)),
            pl.BlockSpec((D, 1), lambda i: (0, 0)),
            pl.BlockSpec((1, 1), lambda i: (0, 0)),
        ],
        out_specs=pl.BlockSpec((_RA, 1), lambda i: (i, 0)),
        out_shape=jax.ShapeDtypeStruct((N, 1), jnp.float32),
    )(nodes, w_gate, b_gate)


def _stats_body(x_hbm, idx_hbm, alpha_hbm, gsum_hbm,
                x_sl, idx_sl, tab, gmax, gsum, comb, alpha_buf,
                stage_max, stage_sum):
    c = lax.axis_index("c")
    s = lax.axis_index("s")
    iota16 = lax.broadcasted_iota(jnp.int32, (16,), 0)

    stat0 = s * _STAT_PER_T
    pltpu.sync_copy(x_hbm.at[pl.ds(stat0, _STAT_PER_T)], x_sl)
    pltpu.sync_copy(idx_hbm.at[pl.ds(stat0, _STAT_PER_T)], idx_sl)

    fmin16 = jnp.full((16,), _FMIN, jnp.float32)

    def _init_tab(v, carry):
        tab[pl.ds(v * 16, 16)] = fmin16
        return carry

    lax.fori_loop(0, S * 16 // 16, _init_tab, 0)

    def _max_step(t, carry):
        x16 = x_sl[pl.ds(t * 16, 16)]
        seg16 = idx_sl[pl.ds(t * 16, 16)]
        addr = seg16 * 16 + iota16
        cur = plsc.load_gather(tab, [addr])
        plsc.store_scatter(tab, [addr], jnp.maximum(cur, x16))
        return carry

    lax.fori_loop(0, _STAT_PER_T // 16, _max_step, 0)

    def _lane_red_max(g, carry):
        segs = (g * 16 + iota16) * 16
        acc = plsc.load_gather(tab, [segs])
        for l in range(1, 16):
            acc = jnp.maximum(acc, plsc.load_gather(tab, [segs + l]))
        gmax[pl.ds(g * 16, 16)] = acc
        return carry

    lax.fori_loop(0, S // 16, _lane_red_max, 0)
    pltpu.sync_copy(gmax, stage_max.at[s])
    plsc.subcore_barrier()
    pltpu.sync_copy(stage_max, comb)

    def _comb_max(g, carry):
        acc = comb[0, pl.ds(g * 16, 16)]
        for t in range(1, _NS):
            acc = jnp.maximum(acc, comb[t, pl.ds(g * 16, 16)])
        gmax[pl.ds(g * 16, 16)] = acc
        return carry

    lax.fori_loop(0, S // 16, _comb_max, 0)

    zero16 = jnp.zeros((16,), jnp.float32)

    def _zero_tab(v, carry):
        tab[pl.ds(v * 16, 16)] = zero16
        return carry

    lax.fori_loop(0, S * 16 // 16, _zero_tab, 0)

    def _sum_step(t, carry):
        x16 = x_sl[pl.ds(t * 16, 16)]
        seg16 = idx_sl[pl.ds(t * 16, 16)]
        m16 = plsc.load_gather(gmax, [seg16])
        e16 = jnp.exp(x16 - m16)
        plsc.addupdate_scatter(tab, [seg16 * 16 + iota16], e16)
        return carry

    lax.fori_loop(0, _STAT_PER_T // 16, _sum_step, 0)

    def _lane_red_sum(g, carry):
        segs = (g * 16 + iota16) * 16
        acc = plsc.load_gather(tab, [segs])
        for l in range(1, 16):
            acc = acc + plsc.load_gather(tab, [segs + l])
        gsum[pl.ds(g * 16, 16)] = acc
        return carry

    lax.fori_loop(0, S // 16, _lane_red_sum, 0)
    pltpu.sync_copy(gsum, stage_sum.at[s])
    plsc.subcore_barrier()
    pltpu.sync_copy(stage_sum, comb)

    def _comb_sum(g, carry):
        acc = comb[0, pl.ds(g * 16, 16)]
        for t in range(1, _NS):
            acc = acc + comb[t, pl.ds(g * 16, 16)]
        gsum[pl.ds(g * 16, 16)] = acc
        return carry

    lax.fori_loop(0, S // 16, _comb_sum, 0)

    @pl.when((c == 0) & (s == 0))
    def _emit_gsum():
        pltpu.sync_copy(gsum, gsum_hbm)

    a0 = (c * _NS + s) * _ALPHA_PER_T
    pltpu.sync_copy(x_hbm.at[pl.ds(a0, _ALPHA_PER_T)], x_sl.at[pl.ds(0, _ALPHA_PER_T)])
    pltpu.sync_copy(idx_hbm.at[pl.ds(a0, _ALPHA_PER_T)], idx_sl.at[pl.ds(0, _ALPHA_PER_T)])

    def _alpha_step(t, carry):
        x16 = x_sl[pl.ds(t * 16, 16)]
        seg16 = idx_sl[pl.ds(t * 16, 16)]
        m16 = plsc.load_gather(gmax, [seg16])
        d16 = plsc.load_gather(gsum, [seg16]) + 1e-16
        alpha_buf[pl.ds(t * 16, 16)] = jnp.exp(x16 - m16) / d16
        return carry

    lax.fori_loop(0, _ALPHA_PER_T // 16, _alpha_step, 0)
    pltpu.sync_copy(alpha_buf, alpha_hbm.at[pl.ds(a0, _ALPHA_PER_T)])


def _stats_call(x_pad, idx_pad):
    mesh = plsc.VectorSubcoreMesh(core_axis_name="c", subcore_axis_name="s",
                                  num_cores=_NC, num_subcores=_NS)
    fn = pl.kernel(
        _stats_body,
        out_type=[
            jax.ShapeDtypeStruct((_NPAD,), jnp.float32),
            jax.ShapeDtypeStruct((S,), jnp.float32),
        ],
        mesh=mesh,
        compiler_params=pltpu.CompilerParams(needs_layout_passes=False),
        scratch_types=[
            pltpu.VMEM((_STAT_PER_T,), jnp.float32),
            pltpu.VMEM((_STAT_PER_T,), jnp.int32),
            pltpu.VMEM((S * 16,), jnp.float32),
            pltpu.VMEM((S,), jnp.float32),
            pltpu.VMEM((S,), jnp.float32),
            pltpu.VMEM((_NS, S), jnp.float32),
            pltpu.VMEM((_ALPHA_PER_T,), jnp.float32),
            pltpu.VMEM_SHARED((_NS, S), jnp.float32),
            pltpu.VMEM_SHARED((_NS, S), jnp.float32),
        ],
    )
    return fn(x_pad, idx_pad)


def _pool_body(nodes_hbm, alpha_hbm, seg_hbm, out_hbm,
               rows0, rows1, alpha0, alpha1, seg0, seg1, ridx0, ridx1,
               acc, sem0, sem1):
    c = lax.axis_index("c")
    s = lax.axis_index("s")
    w = c * _NS + s
    rsh = w % _RSH
    cg = w // _RSH
    col0 = cg * _CGW
    shard0 = rsh * _PER_SH

    zero16 = jnp.zeros((16,), jnp.float32)

    def _zero_row(r, carry):
        for cc in range(_CGW // 16):
            acc[r, pl.ds(cc * 16, 16)] = zero16
        return carry

    lax.fori_loop(0, S, _zero_row, 0)

    iota16 = lax.broadcasted_iota(jnp.int32, (16,), 0)
    bufs = ((rows0, alpha0, seg0, ridx0, sem0),
            (rows1, alpha1, seg1, ridx1, sem1))

    def _issue(b, jc):
        rows_b, alpha_b, seg_b, ridx_b, sem_b = bufs[b]
        base = shard0 + jc * _CH
        for t in range(_CH // 16):
            ridx_b[pl.ds(t * 16, 16)] = jnp.minimum(
                base + t * 16 + iota16, N - 1)
        pltpu.async_copy(alpha_hbm.at[pl.ds(base, _CH)], alpha_b, sem_b)
        pltpu.async_copy(seg_hbm.at[pl.ds(base, _CH)], seg_b, sem_b)
        pltpu.async_copy(nodes_hbm.at[ridx_b, pl.ds(col0, _CGW)],
                         rows_b, sem_b)

    def _wait(b):
        rows_b, alpha_b, seg_b, ridx_b, sem_b = bufs[b]
        pltpu.make_async_copy(alpha_hbm.at[pl.ds(0, _CH)], alpha_b,
                              sem_b).wait()
        pltpu.make_async_copy(seg_hbm.at[pl.ds(0, _CH)], seg_b,
                              sem_b).wait()
        pltpu.make_async_copy(nodes_hbm.at[pl.ds(0, _CH), pl.ds(0, _CGW)],
                              rows_b, sem_b).wait()

    def _process(b):
        rows_b, alpha_b, seg_b, ridx_b, sem_b = bufs[b]

        @plsc.parallel_loop(0, _CH, unroll=8)
        def _row(r):
            r16 = jnp.broadcast_to(r, (16,)).astype(jnp.int32)
            a16 = plsc.load_gather(alpha_b, [r16])
            s16 = plsc.load_gather(seg_b, [r16])
            for cc in range(_CGW // 16):
                val = rows_b[r, pl.ds(cc * 16, 16)] * a16
                plsc.addupdate_scatter(acc, [s16, cc * 16 + iota16], val)

    _issue(0, 0)

    def _pair(k, carry):
        j0 = 2 * k
        _issue(1, j0 + 1)
        _wait(0)
        _process(0)

        @pl.when(j0 + 2 < _NCH)
        def _prefetch():
            _issue(0, j0 + 2)

        _wait(1)
        _process(1)
        return carry

    lax.fori_loop(0, _NCH // 2, _pair, 0)
    pltpu.sync_copy(acc, out_hbm.at[rsh, :, pl.ds(col0, _CGW)])


def _pool_call(nodes, alpha_pad, idx_pad):
    mesh = plsc.VectorSubcoreMesh(core_axis_name="c", subcore_axis_name="s",
                                  num_cores=_NC, num_subcores=_NS)
    fn = pl.kernel(
        _pool_body,
        out_type=jax.ShapeDtypeStruct((_RSH, S, D), jnp.float32),
        mesh=mesh,
        compiler_params=pltpu.CompilerParams(needs_layout_passes=False),
        scratch_types=[
            pltpu.VMEM((_CH, _CGW), jnp.float32),
            pltpu.VMEM((_CH, _CGW), jnp.float32),
            pltpu.VMEM((_CH,), jnp.float32),
            pltpu.VMEM((_CH,), jnp.float32),
            pltpu.VMEM((_CH,), jnp.int32),
            pltpu.VMEM((_CH,), jnp.int32),
            pltpu.VMEM((_CH,), jnp.int32),
            pltpu.VMEM((_CH,), jnp.int32),
            pltpu.VMEM((S, _CGW), jnp.float32),
            pltpu.SemaphoreType.DMA,
            pltpu.SemaphoreType.DMA,
        ],
    )
    return fn(nodes, alpha_pad, idx_pad)


def _final_body(pooled_ref, c_ref, wa_ref, ba_ref, out_ref):
    p = pooled_ref[0]
    for k in range(1, _RSH):
        p = p + pooled_ref[k]
    out_ref[...] = (jnp.dot(p, wa_ref[...], preferred_element_type=jnp.float32)
                    + c_ref[...] * ba_ref[...])


def _final_call(pooled, c_col, w_attn, b_attn_row):
    return pl.pallas_call(
        _final_body,
        out_shape=jax.ShapeDtypeStruct((S, D), jnp.float32),
    )(pooled, c_col, w_attn, b_attn_row)


def kernel(nodes, batch_idx, W_gate, b_gate, W_attn, b_attn):
    idx32 = batch_idx.astype(jnp.int32)
    x = _gate_call(nodes, W_gate, b_gate.reshape(1, 1))
    x_pad = jnp.concatenate(
        [x.reshape(N), jnp.full((_NPAD - N,), -jnp.inf, jnp.float32)])
    idx_pad = jnp.concatenate([idx32, jnp.zeros((_NPAD - N,), jnp.int32)])
    alpha_pad, gsum = _stats_call(x_pad, idx_pad)
    pooled = _pool_call(nodes, alpha_pad, idx_pad)
    gsum_col = gsum.reshape(S, 1)
    c_col = gsum_col / (gsum_col + 1e-16)
    return _final_call(pooled, c_col, W_attn, b_attn.reshape(1, D))


# speedup vs baseline: 1.0532x; 1.0532x over previous
import functools

import jax
import jax.numpy as jnp
from jax import lax
from jax.experimental import pallas as pl
from jax.experimental.pallas import tpu as pltpu
from jax.experimental.pallas import tpu_sc as plsc

N = 50000
D = 512
S = 512

_FMIN = jnp.finfo(jnp.float32).min

_RA = 2000
_NBA = N // _RA

_NC = 2
_NS = 16
_NPAD = 51200
_STAT_PER_T = _NPAD // _NS
_ALPHA_PER_T = _NPAD // (_NC * _NS)

_RSH = 8
_CG = 4
_CGW = D // _CG
_CH = 64
_PER_SH = _NPAD // _RSH
_NCH = _PER_SH // _CH


def _gate_body(nodes_ref, wg_ref, bg_ref, x_ref):
    x_ref[...] = jnp.dot(nodes_ref[...], wg_ref[...],
                         preferred_element_type=jnp.float32) + bg_ref[0, 0]


def _gate_call(nodes, w_gate, b_gate):
    return pl.pallas_call(
        _gate_body,
        grid=(_NBA,),
        in_specs=[
            pl.BlockSpec((_RA, D), lambda i: (i, 0)),
            pl.BlockSpec((D, 1), lambda i: (0, 0)),
            pl.BlockSpec((1, 1), lambda i: (0, 0)),
        ],
        out_specs=pl.BlockSpec((_RA, 1), lambda i: (i, 0)),
        out_shape=jax.ShapeDtypeStruct((N, 1), jnp.float32),
    )(nodes, w_gate, b_gate)


def _stats_body(x_hbm, idx_hbm, alpha_hbm, gsum_hbm,
                x_sl, idx_sl, tab, gmax, gsum, comb, alpha_buf,
                stage_max, stage_sum):
    c = lax.axis_index("c")
    s = lax.axis_index("s")
    iota16 = lax.broadcasted_iota(jnp.int32, (16,), 0)

    stat0 = s * _STAT_PER_T
    pltpu.sync_copy(x_hbm.at[pl.ds(stat0, _STAT_PER_T)], x_sl)
    pltpu.sync_copy(idx_hbm.at[pl.ds(stat0, _STAT_PER_T)], idx_sl)

    fmin16 = jnp.full((16,), _FMIN, jnp.float32)

    def _init_tab(v, carry):
        tab[pl.ds(v * 16, 16)] = fmin16
        return carry

    lax.fori_loop(0, S * 16 // 16, _init_tab, 0)

    def _max_step(t, carry):
        x16 = x_sl[pl.ds(t * 16, 16)]
        seg16 = idx_sl[pl.ds(t * 16, 16)]
        addr = seg16 * 16 + iota16
        cur = plsc.load_gather(tab, [addr])
        plsc.store_scatter(tab, [addr], jnp.maximum(cur, x16))
        return carry

    lax.fori_loop(0, _STAT_PER_T // 16, _max_step, 0)

    def _lane_red_max(g, carry):
        segs = (g * 16 + iota16) * 16
        acc = plsc.load_gather(tab, [segs])
        for l in range(1, 16):
            acc = jnp.maximum(acc, plsc.load_gather(tab, [segs + l]))
        gmax[pl.ds(g * 16, 16)] = acc
        return carry

    lax.fori_loop(0, S // 16, _lane_red_max, 0)
    pltpu.sync_copy(gmax, stage_max.at[s])
    plsc.subcore_barrier()
    pltpu.sync_copy(stage_max, comb)

    def _comb_max(g, carry):
        acc = comb[0, pl.ds(g * 16, 16)]
        for t in range(1, _NS):
            acc = jnp.maximum(acc, comb[t, pl.ds(g * 16, 16)])
        gmax[pl.ds(g * 16, 16)] = acc
        return carry

    lax.fori_loop(0, S // 16, _comb_max, 0)

    zero16 = jnp.zeros((16,), jnp.float32)

    def _zero_tab(v, carry):
        tab[pl.ds(v * 16, 16)] = zero16
        return carry

    lax.fori_loop(0, S * 16 // 16, _zero_tab, 0)

    def _sum_step(t, carry):
        x16 = x_sl[pl.ds(t * 16, 16)]
        seg16 = idx_sl[pl.ds(t * 16, 16)]
        m16 = plsc.load_gather(gmax, [seg16])
        e16 = jnp.exp(x16 - m16)
        plsc.addupdate_scatter(tab, [seg16 * 16 + iota16], e16)
        return carry

    lax.fori_loop(0, _STAT_PER_T // 16, _sum_step, 0)

    def _lane_red_sum(g, carry):
        segs = (g * 16 + iota16) * 16
        acc = plsc.load_gather(tab, [segs])
        for l in range(1, 16):
            acc = acc + plsc.load_gather(tab, [segs + l])
        gsum[pl.ds(g * 16, 16)] = acc
        return carry

    lax.fori_loop(0, S // 16, _lane_red_sum, 0)
    pltpu.sync_copy(gsum, stage_sum.at[s])
    plsc.subcore_barrier()
    pltpu.sync_copy(stage_sum, comb)

    def _comb_sum(g, carry):
        acc = comb[0, pl.ds(g * 16, 16)]
        for t in range(1, _NS):
            acc = acc + comb[t, pl.ds(g * 16, 16)]
        gsum[pl.ds(g * 16, 16)] = acc
        return carry

    lax.fori_loop(0, S // 16, _comb_sum, 0)

    @pl.when((c == 0) & (s == 0))
    def _emit_gsum():
        pltpu.sync_copy(gsum, gsum_hbm)

    a0 = (c * _NS + s) * _ALPHA_PER_T
    pltpu.sync_copy(x_hbm.at[pl.ds(a0, _ALPHA_PER_T)], x_sl.at[pl.ds(0, _ALPHA_PER_T)])
    pltpu.sync_copy(idx_hbm.at[pl.ds(a0, _ALPHA_PER_T)], idx_sl.at[pl.ds(0, _ALPHA_PER_T)])

    def _alpha_step(t, carry):
        x16 = x_sl[pl.ds(t * 16, 16)]
        seg16 = idx_sl[pl.ds(t * 16, 16)]
        m16 = plsc.load_gather(gmax, [seg16])
        d16 = plsc.load_gather(gsum, [seg16]) + 1e-16
        alpha_buf[pl.ds(t * 16, 16)] = jnp.exp(x16 - m16) / d16
        return carry

    lax.fori_loop(0, _ALPHA_PER_T // 16, _alpha_step, 0)
    pltpu.sync_copy(alpha_buf, alpha_hbm.at[pl.ds(a0, _ALPHA_PER_T)])


def _stats_call(x_pad, idx_pad):
    mesh = plsc.VectorSubcoreMesh(core_axis_name="c", subcore_axis_name="s",
                                  num_cores=_NC, num_subcores=_NS)
    fn = pl.kernel(
        _stats_body,
        out_type=[
            jax.ShapeDtypeStruct((_NPAD,), jnp.float32),
            jax.ShapeDtypeStruct((S,), jnp.float32),
        ],
        mesh=mesh,
        compiler_params=pltpu.CompilerParams(needs_layout_passes=False),
        scratch_types=[
            pltpu.VMEM((_STAT_PER_T,), jnp.float32),
            pltpu.VMEM((_STAT_PER_T,), jnp.int32),
            pltpu.VMEM((S * 16,), jnp.float32),
            pltpu.VMEM((S,), jnp.float32),
            pltpu.VMEM((S,), jnp.float32),
            pltpu.VMEM((_NS, S), jnp.float32),
            pltpu.VMEM((_ALPHA_PER_T,), jnp.float32),
            pltpu.VMEM_SHARED((_NS, S), jnp.float32),
            pltpu.VMEM_SHARED((_NS, S), jnp.float32),
        ],
    )
    return fn(x_pad, idx_pad)


def _pool_body(nodes_hbm, alpha_hbm, seg_hbm, out_hbm,
               rows0, rows1, alpha0, alpha1, seg0, seg1, ridx0, ridx1,
               acc, sem0, sem1):
    c = lax.axis_index("c")
    s = lax.axis_index("s")
    w = c * _NS + s
    rsh = w % _RSH
    cg = w // _RSH
    col0 = cg * _CGW
    shard0 = rsh * _PER_SH

    zero16 = jnp.zeros((16,), jnp.float32)

    def _zero_row(r, carry):
        for cc in range(_CGW // 16):
            acc[r, pl.ds(cc * 16, 16)] = zero16
        return carry

    lax.fori_loop(0, S, _zero_row, 0)

    iota16 = lax.broadcasted_iota(jnp.int32, (16,), 0)
    bufs = ((rows0, alpha0, seg0, ridx0, sem0),
            (rows1, alpha1, seg1, ridx1, sem1))

    def _issue(b, jc):
        rows_b, alpha_b, seg_b, ridx_b, sem_b = bufs[b]
        base = shard0 + jc * _CH
        for t in range(_CH // 16):
            ridx_b[pl.ds(t * 16, 16)] = jnp.minimum(
                base + t * 16 + iota16, N - 1)
        pltpu.async_copy(alpha_hbm.at[pl.ds(base, _CH)], alpha_b, sem_b)
        pltpu.async_copy(seg_hbm.at[pl.ds(base, _CH)], seg_b, sem_b)
        pltpu.async_copy(nodes_hbm.at[ridx_b, pl.ds(col0, _CGW)],
                         rows_b, sem_b)

    def _wait(b):
        rows_b, alpha_b, seg_b, ridx_b, sem_b = bufs[b]
        pltpu.make_async_copy(alpha_hbm.at[pl.ds(0, _CH)], alpha_b,
                              sem_b).wait()
        pltpu.make_async_copy(seg_hbm.at[pl.ds(0, _CH)], seg_b,
                              sem_b).wait()
        pltpu.make_async_copy(nodes_hbm.at[pl.ds(0, _CH), pl.ds(0, _CGW)],
                              rows_b, sem_b).wait()

    def _process(b):
        rows_b, alpha_b, seg_b, ridx_b, sem_b = bufs[b]

        @plsc.parallel_loop(0, _CH // 2, unroll=4)
        def _row(r):
            for half in range(2):
                rr = r + half * (_CH // 2)
                r16 = jnp.broadcast_to(rr, (16,)).astype(jnp.int32)
                a16 = plsc.load_gather(alpha_b, [r16])
                s16 = plsc.load_gather(seg_b, [r16])
                for cc in range(_CGW // 16):
                    val = rows_b[rr, pl.ds(cc * 16, 16)] * a16
                    plsc.addupdate_scatter(acc, [s16, cc * 16 + iota16], val)

    _issue(0, 0)

    def _pair(k, carry):
        j0 = 2 * k
        _issue(1, j0 + 1)
        _wait(0)
        _process(0)

        @pl.when(j0 + 2 < _NCH)
        def _prefetch():
            _issue(0, j0 + 2)

        _wait(1)
        _process(1)
        return carry

    lax.fori_loop(0, _NCH // 2, _pair, 0)
    pltpu.sync_copy(acc, out_hbm.at[rsh, :, pl.ds(col0, _CGW)])


def _pool_call(nodes, alpha_pad, idx_pad):
    mesh = plsc.VectorSubcoreMesh(core_axis_name="c", subcore_axis_name="s",
                                  num_cores=_NC, num_subcores=_NS)
    fn = pl.kernel(
        _pool_body,
        out_type=jax.ShapeDtypeStruct((_RSH, S, D), jnp.float32),
        mesh=mesh,
        compiler_params=pltpu.CompilerParams(needs_layout_passes=False),
        scratch_types=[
            pltpu.VMEM((_CH, _CGW), jnp.float32),
            pltpu.VMEM((_CH, _CGW), jnp.float32),
            pltpu.VMEM((_CH,), jnp.float32),
            pltpu.VMEM((_CH,), jnp.float32),
            pltpu.VMEM((_CH,), jnp.int32),
            pltpu.VMEM((_CH,), jnp.int32),
            pltpu.VMEM((_CH,), jnp.int32),
            pltpu.VMEM((_CH,), jnp.int32),
            pltpu.VMEM((S, _CGW), jnp.float32),
            pltpu.SemaphoreType.DMA,
            pltpu.SemaphoreType.DMA,
        ],
    )
    return fn(nodes, alpha_pad, idx_pad)


def _final_body(pooled_ref, c_ref, wa_ref, ba_ref, out_ref):
    p = pooled_ref[0]
    for k in range(1, _RSH):
        p = p + pooled_ref[k]
    out_ref[...] = (jnp.dot(p, wa_ref[...], preferred_element_type=jnp.float32)
                    + c_ref[...] * ba_ref[...])


def _final_call(pooled, c_col, w_attn, b_attn_row):
    return pl.pallas_call(
        _final_body,
        out_shape=jax.ShapeDtypeStruct((S, D), jnp.float32),
    )(pooled, c_col, w_attn, b_attn_row)


def kernel(nodes, batch_idx, W_gate, b_gate, W_attn, b_attn):
    idx32 = batch_idx.astype(jnp.int32)
    x = _gate_call(nodes, W_gate, b_gate.reshape(1, 1))
    x_pad = jnp.concatenate(
        [x.reshape(N), jnp.full((_NPAD - N,), -jnp.inf, jnp.float32)])
    idx_pad = jnp.concatenate([idx32, jnp.zeros((_NPAD - N,), jnp.int32)])
    alpha_pad, gsum = _stats_call(x_pad, idx_pad)
    pooled = _pool_call(nodes, alpha_pad, idx_pad)
    gsum_col = gsum.reshape(S, 1)
    c_col = gsum_col / (gsum_col + 1e-16)
    return _final_call(pooled, c_col, W_attn, b_attn.reshape(1, D))


# speedup vs baseline: 1.0576x; 1.0042x over previous
import functools

import jax
import jax.numpy as jnp
from jax import lax
from jax.experimental import pallas as pl
from jax.experimental.pallas import tpu as pltpu
from jax.experimental.pallas import tpu_sc as plsc

N = 50000
D = 512
S = 512

_FMIN = jnp.finfo(jnp.float32).min

_RA = 2000
_NBA = N // _RA

_NC = 2
_NS = 16
_NPAD = 51200
_STAT_PER_T = _NPAD // _NS
_ALPHA_PER_T = _NPAD // (_NC * _NS)

_RSH = 8
_CG = 4
_CGW = D // _CG
_CH = 64
_PER_SH = _NPAD // _RSH
_NCH = _PER_SH // _CH


def _gate_body(nodes_ref, wg_ref, bg_ref, x_ref):
    x_ref[...] = jnp.dot(nodes_ref[...], wg_ref[...],
                         preferred_element_type=jnp.float32) + bg_ref[0, 0]


def _gate_call(nodes, w_gate, b_gate):
    return pl.pallas_call(
        _gate_body,
        grid=(_NBA,),
        in_specs=[
            pl.BlockSpec((_RA, D), lambda i: (i, 0)),
            pl.BlockSpec((D, 1), lambda i: (0, 0)),
            pl.BlockSpec((1, 1), lambda i: (0, 0)),
        ],
        out_specs=pl.BlockSpec((_RA, 1), lambda i: (i, 0)),
        out_shape=jax.ShapeDtypeStruct((N, 1), jnp.float32),
    )(nodes, w_gate, b_gate)


def _stats_body(x_hbm, idx_hbm, alpha_hbm, gsum_hbm,
                x_sl, idx_sl, tab, gmax, gsum, comb, alpha_buf,
                stage_max, stage_sum):
    c = lax.axis_index("c")
    s = lax.axis_index("s")
    iota16 = lax.broadcasted_iota(jnp.int32, (16,), 0)

    stat0 = s * _STAT_PER_T
    pltpu.sync_copy(x_hbm.at[pl.ds(stat0, _STAT_PER_T)], x_sl)
    pltpu.sync_copy(idx_hbm.at[pl.ds(stat0, _STAT_PER_T)], idx_sl)

    fmin16 = jnp.full((16,), _FMIN, jnp.float32)

    def _init_tab(v, carry):
        tab[pl.ds(v * 16, 16)] = fmin16
        return carry

    lax.fori_loop(0, S * 16 // 16, _init_tab, 0)

    def _max_step(t, carry):
        x16 = x_sl[pl.ds(t * 16, 16)]
        seg16 = idx_sl[pl.ds(t * 16, 16)]
        addr = seg16 * 16 + iota16
        cur = plsc.load_gather(tab, [addr])
        plsc.store_scatter(tab, [addr], jnp.maximum(cur, x16))
        return carry

    lax.fori_loop(0, _STAT_PER_T // 16, _max_step, 0)

    def _lane_red_max(g, carry):
        segs = (g * 16 + iota16) * 16
        acc = plsc.load_gather(tab, [segs])
        for l in range(1, 16):
            acc = jnp.maximum(acc, plsc.load_gather(tab, [segs + l]))
        gmax[pl.ds(g * 16, 16)] = acc
        return carry

    lax.fori_loop(0, S // 16, _lane_red_max, 0)
    pltpu.sync_copy(gmax, stage_max.at[s])
    plsc.subcore_barrier()
    pltpu.sync_copy(stage_max, comb)

    def _comb_max(g, carry):
        acc = comb[0, pl.ds(g * 16, 16)]
        for t in range(1, _NS):
            acc = jnp.maximum(acc, comb[t, pl.ds(g * 16, 16)])
        gmax[pl.ds(g * 16, 16)] = acc
        return carry

    lax.fori_loop(0, S // 16, _comb_max, 0)

    zero16 = jnp.zeros((16,), jnp.float32)

    def _zero_tab(v, carry):
        tab[pl.ds(v * 16, 16)] = zero16
        return carry

    lax.fori_loop(0, S * 16 // 16, _zero_tab, 0)

    def _sum_step(t, carry):
        x16 = x_sl[pl.ds(t * 16, 16)]
        seg16 = idx_sl[pl.ds(t * 16, 16)]
        m16 = plsc.load_gather(gmax, [seg16])
        e16 = jnp.exp(x16 - m16)
        plsc.addupdate_scatter(tab, [seg16 * 16 + iota16], e16)
        return carry

    lax.fori_loop(0, _STAT_PER_T // 16, _sum_step, 0)

    def _lane_red_sum(g, carry):
        segs = (g * 16 + iota16) * 16
        acc = plsc.load_gather(tab, [segs])
        for l in range(1, 16):
            acc = acc + plsc.load_gather(tab, [segs + l])
        gsum[pl.ds(g * 16, 16)] = acc
        return carry

    lax.fori_loop(0, S // 16, _lane_red_sum, 0)
    pltpu.sync_copy(gsum, stage_sum.at[s])
    plsc.subcore_barrier()
    pltpu.sync_copy(stage_sum, comb)

    def _comb_sum(g, carry):
        acc = comb[0, pl.ds(g * 16, 16)]
        for t in range(1, _NS):
            acc = acc + comb[t, pl.ds(g * 16, 16)]
        gsum[pl.ds(g * 16, 16)] = acc
        return carry

    lax.fori_loop(0, S // 16, _comb_sum, 0)

    @pl.when((c == 0) & (s == 0))
    def _emit_gsum():
        pltpu.sync_copy(gsum, gsum_hbm)

    a0 = (c * _NS + s) * _ALPHA_PER_T
    pltpu.sync_copy(x_hbm.at[pl.ds(a0, _ALPHA_PER_T)], x_sl.at[pl.ds(0, _ALPHA_PER_T)])
    pltpu.sync_copy(idx_hbm.at[pl.ds(a0, _ALPHA_PER_T)], idx_sl.at[pl.ds(0, _ALPHA_PER_T)])

    def _alpha_step(t, carry):
        x16 = x_sl[pl.ds(t * 16, 16)]
        seg16 = idx_sl[pl.ds(t * 16, 16)]
        m16 = plsc.load_gather(gmax, [seg16])
        d16 = plsc.load_gather(gsum, [seg16]) + 1e-16
        alpha_buf[pl.ds(t * 16, 16)] = jnp.exp(x16 - m16) / d16
        return carry

    lax.fori_loop(0, _ALPHA_PER_T // 16, _alpha_step, 0)
    pltpu.sync_copy(alpha_buf, alpha_hbm.at[pl.ds(a0, _ALPHA_PER_T)])


def _stats_call(x_pad, idx_pad):
    mesh = plsc.VectorSubcoreMesh(core_axis_name="c", subcore_axis_name="s",
                                  num_cores=_NC, num_subcores=_NS)
    fn = pl.kernel(
        _stats_body,
        out_type=[
            jax.ShapeDtypeStruct((_NPAD,), jnp.float32),
            jax.ShapeDtypeStruct((S,), jnp.float32),
        ],
        mesh=mesh,
        compiler_params=pltpu.CompilerParams(needs_layout_passes=False),
        scratch_types=[
            pltpu.VMEM((_STAT_PER_T,), jnp.float32),
            pltpu.VMEM((_STAT_PER_T,), jnp.int32),
            pltpu.VMEM((S * 16,), jnp.float32),
            pltpu.VMEM((S,), jnp.float32),
            pltpu.VMEM((S,), jnp.float32),
            pltpu.VMEM((_NS, S), jnp.float32),
            pltpu.VMEM((_ALPHA_PER_T,), jnp.float32),
            pltpu.VMEM_SHARED((_NS, S), jnp.float32),
            pltpu.VMEM_SHARED((_NS, S), jnp.float32),
        ],
    )
    return fn(x_pad, idx_pad)


def _pool_body(nodes_hbm, alpha_hbm, seg_hbm, out_hbm,
               rows0, rows1, alpha0, alpha1, seg0, seg1, ridx0, ridx1,
               acc, sem0, sem1):
    c = lax.axis_index("c")
    s = lax.axis_index("s")
    w = c * _NS + s
    rsh = w % _RSH
    cg = w // _RSH
    col0 = cg * _CGW
    shard0 = rsh * _PER_SH

    zero16 = jnp.zeros((16,), jnp.float32)

    def _zero_row(r, carry):
        for cc in range(_CGW // 16):
            acc[r, pl.ds(cc * 16, 16)] = zero16
        return carry

    lax.fori_loop(0, S, _zero_row, 0)

    iota16 = lax.broadcasted_iota(jnp.int32, (16,), 0)
    bufs = ((rows0, alpha0, seg0, ridx0, sem0),
            (rows1, alpha1, seg1, ridx1, sem1))

    def _issue(b, jc):
        rows_b, alpha_b, seg_b, ridx_b, sem_b = bufs[b]
        base = shard0 + jc * _CH
        pltpu.async_copy(alpha_hbm.at[pl.ds(base, _CH)], alpha_b, sem_b)
        pltpu.async_copy(seg_hbm.at[pl.ds(base, _CH)], seg_b, sem_b)

        @pl.when(base + _CH <= N)
        def _linear():
            pltpu.async_copy(nodes_hbm.at[pl.ds(base, _CH), pl.ds(col0, _CGW)],
                             rows_b, sem_b)

        @pl.when(base + _CH > N)
        def _indirect():
            for t in range(_CH // 16):
                ridx_b[pl.ds(t * 16, 16)] = jnp.minimum(
                    base + t * 16 + iota16, N - 1)
            pltpu.async_copy(nodes_hbm.at[ridx_b, pl.ds(col0, _CGW)],
                             rows_b, sem_b)

    def _wait(b):
        rows_b, alpha_b, seg_b, ridx_b, sem_b = bufs[b]
        pltpu.make_async_copy(alpha_hbm.at[pl.ds(0, _CH)], alpha_b,
                              sem_b).wait()
        pltpu.make_async_copy(seg_hbm.at[pl.ds(0, _CH)], seg_b,
                              sem_b).wait()
        pltpu.make_async_copy(nodes_hbm.at[pl.ds(0, _CH), pl.ds(0, _CGW)],
                              rows_b, sem_b).wait()

    def _process(b):
        rows_b, alpha_b, seg_b, ridx_b, sem_b = bufs[b]

        @plsc.parallel_loop(0, _CH // 2, unroll=4)
        def _row(r):
            for half in range(2):
                rr = r + half * (_CH // 2)
                r16 = jnp.broadcast_to(rr, (16,)).astype(jnp.int32)
                a16 = plsc.load_gather(alpha_b, [r16])
                s16 = plsc.load_gather(seg_b, [r16])
                for cc in range(_CGW // 16):
                    val = rows_b[rr, pl.ds(cc * 16, 16)] * a16
                    plsc.addupdate_scatter(acc, [s16, cc * 16 + iota16], val)

    _issue(0, 0)

    def _pair(k, carry):
        j0 = 2 * k
        _issue(1, j0 + 1)
        _wait(0)
        _process(0)

        @pl.when(j0 + 2 < _NCH)
        def _prefetch():
            _issue(0, j0 + 2)

        _wait(1)
        _process(1)
        return carry

    lax.fori_loop(0, _NCH // 2, _pair, 0)
    pltpu.sync_copy(acc, out_hbm.at[rsh, :, pl.ds(col0, _CGW)])


def _pool_call(nodes, alpha_pad, idx_pad):
    mesh = plsc.VectorSubcoreMesh(core_axis_name="c", subcore_axis_name="s",
                                  num_cores=_NC, num_subcores=_NS)
    fn = pl.kernel(
        _pool_body,
        out_type=jax.ShapeDtypeStruct((_RSH, S, D), jnp.float32),
        mesh=mesh,
        compiler_params=pltpu.CompilerParams(needs_layout_passes=False),
        scratch_types=[
            pltpu.VMEM((_CH, _CGW), jnp.float32),
            pltpu.VMEM((_CH, _CGW), jnp.float32),
            pltpu.VMEM((_CH,), jnp.float32),
            pltpu.VMEM((_CH,), jnp.float32),
            pltpu.VMEM((_CH,), jnp.int32),
            pltpu.VMEM((_CH,), jnp.int32),
            pltpu.VMEM((_CH,), jnp.int32),
            pltpu.VMEM((_CH,), jnp.int32),
            pltpu.VMEM((S, _CGW), jnp.float32),
            pltpu.SemaphoreType.DMA,
            pltpu.SemaphoreType.DMA,
        ],
    )
    return fn(nodes, alpha_pad, idx_pad)


def _final_body(pooled_ref, c_ref, wa_ref, ba_ref, out_ref):
    p = pooled_ref[0]
    for k in range(1, _RSH):
        p = p + pooled_ref[k]
    out_ref[...] = (jnp.dot(p, wa_ref[...], preferred_element_type=jnp.float32)
                    + c_ref[...] * ba_ref[...])


def _final_call(pooled, c_col, w_attn, b_attn_row):
    return pl.pallas_call(
        _final_body,
        out_shape=jax.ShapeDtypeStruct((S, D), jnp.float32),
    )(pooled, c_col, w_attn, b_attn_row)


def kernel(nodes, batch_idx, W_gate, b_gate, W_attn, b_attn):
    idx32 = batch_idx.astype(jnp.int32)
    x = _gate_call(nodes, W_gate, b_gate.reshape(1, 1))
    x_pad = jnp.concatenate(
        [x.reshape(N), jnp.full((_NPAD - N,), -jnp.inf, jnp.float32)])
    idx_pad = jnp.concatenate([idx32, jnp.zeros((_NPAD - N,), jnp.int32)])
    alpha_pad, gsum = _stats_call(x_pad, idx_pad)
    pooled = _pool_call(nodes, alpha_pad, idx_pad)
    gsum_col = gsum.reshape(S, 1)
    c_col = gsum_col / (gsum_col + 1e-16)
    return _final_call(pooled, c_col, W_attn, b_attn.reshape(1, D))
